# Initial kernel scaffold; baseline (speedup 1.0000x reference)
import jax
import jax.numpy as jnp
from jax.experimental import pallas as pl


def kernel(node_type_ids, text_encodings, edge_index, edge_type_ids, node_type_table, edge_type_table, W, b):
    raise NotImplementedError("write your pallas kernel here")


# speedup vs baseline: 2.1196x; 2.1196x over previous
import functools

import jax
import jax.numpy as jnp
from jax import lax
from jax.experimental import pallas as pl
from jax.experimental.pallas import tpu as pltpu
from jax.experimental.pallas import tpu_sc as plsc

N_NODES = 10000
N_EDGES = 160000
TEXT_REP = 256
NODE_TYPE_EMB = 128
EDGE_TYPE_EMB = 16
NODE_HIDDEN = 256
NUM_NODE_TYPES = 16

_NC = 2
_NS = 16
_NW = _NC * _NS
_CH = 128
_K = 40
_EPAD = _NW * _K * _CH

_RB = 2000
_G = N_NODES // _RB


def _node_body(ids_ref, text_ref, ntt_ref, w_ref, b_ref, out_ref):
    ft = jnp.dot(ntt_ref[:], w_ref[:NODE_TYPE_EMB, :],
                 preferred_element_type=jnp.float32) + b_ref[:]
    ids = ids_ref[0, 0, :]
    onehot = (ids[:, None] == lax.broadcasted_iota(
        jnp.int32, (_RB, NUM_NODE_TYPES), 1)).astype(jnp.float32)
    acc = jnp.dot(text_ref[:], w_ref[NODE_TYPE_EMB:, :],
                  preferred_element_type=jnp.float32)
    out_ref[:] = acc + jnp.dot(onehot, ft,
                               preferred_element_type=jnp.float32)


def _node_proj(ids3, text, ntt, w, b2):
    return pl.pallas_call(
        _node_body,
        grid=(_G,),
        in_specs=[
            pl.BlockSpec((1, 1, _RB), lambda i: (i, 0, 0)),
            pl.BlockSpec((_RB, TEXT_REP), lambda i: (i, 0)),
            pl.BlockSpec((NUM_NODE_TYPES, NODE_TYPE_EMB), lambda i: (0, 0)),
            pl.BlockSpec((NODE_TYPE_EMB + TEXT_REP, NODE_HIDDEN),
                         lambda i: (0, 0)),
            pl.BlockSpec((1, NODE_HIDDEN), lambda i: (0, 0)),
        ],
        out_specs=pl.BlockSpec((_RB, NODE_HIDDEN), lambda i: (i, 0)),
        out_shape=jax.ShapeDtypeStruct((N_NODES, NODE_HIDDEN), jnp.float32),
    )(ids3, text, ntt, w, b2)


_EPW = _EPAD // _NW
_NG = _EPW // 16


def _edge_gather(table, ids2):
    mesh = plsc.VectorSubcoreMesh(core_axis_name="c", subcore_axis_name="s")

    @functools.partial(
        pl.kernel, mesh=mesh,
        compiler_params=pltpu.CompilerParams(
            needs_layout_passes=False, use_tc_tiling_on_sc=False),
        out_type=jax.ShapeDtypeStruct((_EPAD, EDGE_TYPE_EMB), jnp.float32),
        scratch_types=[
            pltpu.VMEM((8, EDGE_TYPE_EMB), jnp.float32),
            pltpu.VMEM((_EPW,), jnp.int32),
            pltpu.VMEM((_EPW, EDGE_TYPE_EMB), jnp.float32),
        ],
    )
    def k(table_hbm, idx_hbm, out_hbm, table_v, idx_v, out_v):
        wid = lax.axis_index("s") * _NC + lax.axis_index("c")
        pltpu.sync_copy(table_hbm, table_v)
        pltpu.sync_copy(idx_hbm.at[wid], idx_v)
        lane = lax.broadcasted_iota(jnp.int32, (16,), 0)

        def body(j, carry):
            ids16 = idx_v[pl.ds(j * 16, 16)]
            rows = j * 16 + lane
            for d in range(EDGE_TYPE_EMB):
                col = jnp.full((16,), d, jnp.int32)
                vals = plsc.load_gather(table_v, [ids16, col])
                plsc.store_scatter(out_v, [rows, col], vals)
            return carry

        lax.fori_loop(0, _NG, body, 0)
        pltpu.sync_copy(out_v, out_hbm.at[pl.ds(wid * _EPW, _EPW)])

    return k(table, ids2)


def kernel(node_type_ids, text_encodings, edge_index, edge_type_ids,
           node_type_table, edge_type_table, W, b):
    del edge_index
    ids3 = node_type_ids.astype(jnp.int32).reshape(_G, 1, _RB)
    b2 = b.reshape(1, NODE_HIDDEN)
    node_h = _node_proj(ids3, text_encodings, node_type_table, W, b2)
    eidx = edge_type_ids.astype(jnp.int32)
    eidx2 = jnp.concatenate(
        [eidx, jnp.zeros((_EPAD - N_EDGES,), jnp.int32)]).reshape(_NW, _EPW)
    edge_h = _edge_gather(edge_type_table, eidx2)[:N_EDGES]
    return node_h, edge_h


# speedup vs baseline: 3.4968x; 1.6498x over previous
import functools

import jax
import jax.numpy as jnp
from jax import lax
from jax.experimental import pallas as pl
from jax.experimental.pallas import tpu as pltpu
from jax.experimental.pallas import tpu_sc as plsc

N_NODES = 10000
N_EDGES = 160000
TEXT_REP = 256
NODE_TYPE_EMB = 128
EDGE_TYPE_EMB = 16
NODE_HIDDEN = 256
NUM_NODE_TYPES = 16

_NC = 2
_NS = 16
_NW = _NC * _NS
_EPW = N_EDGES // _NW
_EPWP = _EPW + 16 - (_EPW % 16)
_NG = (_EPW + 15) // 16

_RB = 2000
_G = N_NODES // _RB


def _node_body(ids_ref, text_ref, ntt_ref, w_ref, b_ref, out_ref):
    ft = jnp.dot(ntt_ref[:], w_ref[:NODE_TYPE_EMB, :],
                 preferred_element_type=jnp.float32) + b_ref[:]
    ids = ids_ref[0, 0, :]
    onehot = (ids[:, None] == lax.broadcasted_iota(
        jnp.int32, (_RB, NUM_NODE_TYPES), 1)).astype(jnp.float32)
    acc = jnp.dot(text_ref[:], w_ref[NODE_TYPE_EMB:, :],
                  preferred_element_type=jnp.float32)
    out_ref[:] = acc + jnp.dot(onehot, ft,
                               preferred_element_type=jnp.float32)


def _node_proj(ids3, text, ntt, w, b2):
    return pl.pallas_call(
        _node_body,
        grid=(_G,),
        in_specs=[
            pl.BlockSpec((1, 1, _RB), lambda i: (i, 0, 0)),
            pl.BlockSpec((_RB, TEXT_REP), lambda i: (i, 0)),
            pl.BlockSpec((NUM_NODE_TYPES, NODE_TYPE_EMB), lambda i: (0, 0)),
            pl.BlockSpec((NODE_TYPE_EMB + TEXT_REP, NODE_HIDDEN),
                         lambda i: (0, 0)),
            pl.BlockSpec((1, NODE_HIDDEN), lambda i: (0, 0)),
        ],
        out_specs=pl.BlockSpec((_RB, NODE_HIDDEN), lambda i: (i, 0)),
        out_shape=jax.ShapeDtypeStruct((N_NODES, NODE_HIDDEN), jnp.float32),
    )(ids3, text, ntt, w, b2)


def _edge_gather(table, ids):
    mesh = plsc.VectorSubcoreMesh(core_axis_name="c", subcore_axis_name="s")

    @functools.partial(
        pl.kernel, mesh=mesh,
        compiler_params=pltpu.CompilerParams(
            needs_layout_passes=False, use_tc_tiling_on_sc=False),
        out_type=jax.ShapeDtypeStruct((N_EDGES, EDGE_TYPE_EMB), jnp.float32),
        scratch_types=[
            pltpu.VMEM((8, EDGE_TYPE_EMB), jnp.float32),
            pltpu.VMEM((_EPWP,), jnp.int32),
            pltpu.VMEM((_EPWP, EDGE_TYPE_EMB), jnp.float32),
        ],
    )
    def k(table_hbm, idx_hbm, out_hbm, table_v, idx_v, out_v):
        wid = lax.axis_index("s") * _NC + lax.axis_index("c")
        lane = lax.broadcasted_iota(jnp.int32, (16,), 0)
        idx_v[pl.ds(_EPWP - 16, 16)] = jnp.zeros((16,), jnp.int32)
        pltpu.sync_copy(table_hbm, table_v)
        pltpu.sync_copy(idx_hbm.at[pl.ds(wid * _EPW, _EPW)], idx_v.at[pl.ds(0, _EPW)])
        cols = [jnp.full((16,), d, jnp.int32) for d in range(EDGE_TYPE_EMB)]

        def body(j, carry):
            ids16 = idx_v[pl.ds(j * 16, 16)]
            rows = j * 16 + lane
            vals = [plsc.load_gather(table_v, [ids16, cols[d]])
                    for d in range(EDGE_TYPE_EMB)]
            for d in range(EDGE_TYPE_EMB):
                plsc.store_scatter(out_v, [rows, cols[d]], vals[d])
            return carry

        lax.fori_loop(0, _NG, body, 0)
        pltpu.sync_copy(out_v.at[pl.ds(0, _EPW)],
                        out_hbm.at[pl.ds(wid * _EPW, _EPW)])

    return k(table, ids)


def kernel(node_type_ids, text_encodings, edge_index, edge_type_ids,
           node_type_table, edge_type_table, W, b):
    del edge_index
    ids3 = node_type_ids.astype(jnp.int32).reshape(_G, 1, _RB)
    b2 = b.reshape(1, NODE_HIDDEN)
    node_h = _node_proj(ids3, text_encodings, node_type_table, W, b2)
    edge_h = _edge_gather(edge_type_table, edge_type_ids.astype(jnp.int32))
    return node_h, edge_h
